# baseline (device time: 13907 ns/iter reference)
import jax
import jax.numpy as jnp
from jax import lax
from jax.experimental import pallas as pl
from jax.experimental.pallas import tpu as pltpu

N_DEV = 8
N_PAIR = N_DEV // 2


def kernel(x, w_mat):
    m_per, k = x.shape
    n = w_mat.shape[1]
    n_per = n // N_DEV
    bw = 2 * n_per

    def body(x_ref, w_ref, out_ref, comm_ref, send_sems, recv_sems):
        my = lax.axis_index("i")
        my_half = my // 2
        my_par = lax.rem(my, 2)
        partner = my + 1 - 2 * my_par

        barrier_sem = pltpu.get_barrier_semaphore()
        for o in range(1, N_DEV):
            peer = lax.rem(my + o, N_DEV)
            pl.semaphore_signal(
                barrier_sem, inc=1,
                device_id=(peer,), device_id_type=pl.DeviceIdType.MESH,
            )
        pl.semaphore_wait(barrier_sem, N_DEV - 1)

        x_blk = x_ref[:, :]
        sends = []

        def send_half(slot, half_start, dest, sem_idx):
            rdma = pltpu.make_async_remote_copy(
                src_ref=comm_ref.at[slot, :, pl.ds(half_start, n_per)],
                dst_ref=out_ref.at[pl.ds(my * m_per, m_per)],
                send_sem=send_sems.at[sem_idx],
                recv_sem=recv_sems.at[my],
                device_id=(dest,),
                device_id_type=pl.DeviceIdType.MESH,
            )
            rdma.start()
            sends.append(rdma)

        for b in range(N_PAIR):
            c = lax.rem(my_half + b, N_PAIR)
            comm_ref[b, :, :] = jnp.dot(
                x_blk,
                w_ref[:, pl.ds(c * bw, bw)],
                preferred_element_type=jnp.float32,
            )
            if b == 0:
                send_half(0, (1 - my_par) * n_per, partner, 7)
                out_ref[pl.ds(my * m_per, m_per), :] = comm_ref[
                    0, :, pl.ds(my_par * n_per, n_per)
                ]
            else:
                send_half(b, 0, 2 * c, 2 * b - 1)
                send_half(b, n_per, 2 * c + 1, 2 * b)

        def wait_from(src):
            recv = pltpu.make_async_remote_copy(
                src_ref=comm_ref.at[0, :, pl.ds(0, n_per)],
                dst_ref=out_ref.at[pl.ds(src * m_per, m_per)],
                send_sem=send_sems.at[0],
                recv_sem=recv_sems.at[src],
                device_id=(src,),
                device_id_type=pl.DeviceIdType.MESH,
            )
            recv.wait_recv()

        wait_from(partner)
        for d in range(1, N_PAIR):
            src_half = lax.rem(my_half + N_PAIR - d, N_PAIR)
            wait_from(2 * src_half)
            wait_from(2 * src_half + 1)

        for rdma in sends:
            rdma.wait_send()

    out_shape = jax.ShapeDtypeStruct((N_DEV * m_per, n_per), jnp.float32)
    return pl.pallas_call(
        body,
        out_shape=out_shape,
        in_specs=[
            pl.BlockSpec(memory_space=pltpu.VMEM),
            pl.BlockSpec(memory_space=pltpu.VMEM),
        ],
        out_specs=pl.BlockSpec(memory_space=pltpu.VMEM),
        scratch_shapes=[
            pltpu.VMEM((N_PAIR, m_per, bw), jnp.float32),
            pltpu.SemaphoreType.DMA((N_DEV,)),
            pltpu.SemaphoreType.DMA((N_DEV,)),
        ],
        compiler_params=pltpu.CompilerParams(collective_id=0),
    )(x, w_mat)
